# baseline (speedup 1.0000x reference)
import jax
import jax.numpy as jnp
from jax.experimental import pallas as pl
from jax.experimental.pallas import tpu as pltpu

_K = 5
_NEG = -3.0e38


def _topk_native_kernel(x_ref, w_ref, o_ref, *ts_refs):
    i = pl.program_id(0)
    nw = x_ref.shape[1]

    @pl.when(i == 0)
    def _():
        neg = jnp.full(ts_refs[0].shape, _NEG, jnp.float32)
        for r in ts_refs:
            r[...] = neg

    ts = [r[...] for r in ts_refs]

    for k in range(nw):
        t = x_ref[0, k]
        for t_i in range(_K):
            cur = ts[t_i]
            hi = jnp.maximum(cur, t)
            t = jnp.minimum(cur, t)
            ts[t_i] = hi

    for r, t in zip(ts_refs, ts):
        r[...] = t

    @pl.when(i == pl.num_programs(0) - 1)
    def _():
        acc = ts[0]
        for t in ts[1:]:
            acc = acc + t
        sim = acc * (1.0 / _K)
        o_ref[...] = jax.lax.dot_general(
            sim, w_ref[...], (((0,), (1,)), ((), ())),
            precision=jax.lax.Precision.HIGHEST,
            preferred_element_type=jnp.float32)


def kernel(prototype_activations, W):
    b, p, h, w = prototype_activations.shape
    c = W.shape[0]
    xt = jnp.transpose(prototype_activations, (2, 3, 1, 0))

    out = pl.pallas_call(
        _topk_native_kernel,
        grid=(h,),
        in_specs=[
            pl.BlockSpec((1, w, p, b), lambda i: (i, 0, 0, 0)),
            pl.BlockSpec((c, p), lambda i: (0, 0)),
        ],
        out_specs=pl.BlockSpec((b, c), lambda i: (0, 0)),
        out_shape=jax.ShapeDtypeStruct((b, c), jnp.float32),
        scratch_shapes=[pltpu.VMEM((p, b), jnp.float32) for _ in range(_K)],
    )(xt, W)
    return out
